# Initial kernel scaffold; baseline (speedup 1.0000x reference)
import jax
import jax.numpy as jnp
from jax.experimental import pallas as pl


def kernel(x, edge_index, W1, b1, W2, b2, Wl, bl):
    raise NotImplementedError("write your pallas kernel here")


# speedup vs baseline: 153.5461x; 153.5461x over previous
import functools

import jax
import jax.numpy as jnp
from jax import lax
from jax.experimental import pallas as pl
from jax.experimental.pallas import tpu as pltpu
from jax.experimental.pallas import tpu_sc as plsc

N = 100000
NP = 100352
NC, NS, L = 2, 16, 16
NW = NC * NS
G = 32
ROWS_PER_W = 800
BLKS = ROWS_PER_W // G
EP = ROWS_PER_W * 128 * NW
STRIPE = NP // NS

_f32 = jnp.float32
_mesh = plsc.VectorSubcoreMesh(core_axis_name="c", subcore_axis_name="s")
_sc_params = pltpu.CompilerParams(needs_layout_passes=False)


def _zero_acc_stripe(zbuf, acc, sid):
    zeros = jnp.zeros((L,), _f32)

    @pl.loop(0, STRIPE // L)
    def _(i):
        zbuf[pl.ds(i * L, L)] = zeros

    pltpu.sync_copy(zbuf, acc.at[pl.ds(sid * STRIPE, STRIPE)])


def _copy_out_stripe(acc, out_hbm, cid, sid):
    pltpu.sync_copy(
        acc.at[pl.ds(sid * STRIPE, STRIPE)],
        out_hbm.at[cid, pl.ds(sid * STRIPE, STRIPE)],
    )


@functools.partial(
    pl.kernel,
    out_type=jax.ShapeDtypeStruct((NC, NP), _f32),
    mesh=_mesh,
    scratch_types=[
        pltpu.VMEM((G, 128), jnp.int32),
        pltpu.VMEM((G, 128), _f32),
        pltpu.VMEM((STRIPE,), _f32),
        pltpu.VMEM_SHARED((NP,), _f32),
    ],
    compiler_params=_sc_params,
)
def _deg_kernel(dst_hbm, out_hbm, dst_v, msg_v, zbuf, acc):
    cid = lax.axis_index("c")
    sid = lax.axis_index("s")
    wid = cid * NS + sid

    _zero_acc_stripe(zbuf, acc, sid)

    ones = jnp.ones((L,), _f32)

    @pl.loop(0, G * 128 // L)
    def _(i):
        msg_v[i // 8, pl.ds((i % 8) * L, L)] = ones

    plsc.subcore_barrier()

    @pl.loop(0, BLKS)
    def _(blk):
        row0 = wid * ROWS_PER_W + blk * G
        pltpu.sync_copy(dst_hbm.at[pl.ds(row0, G)], dst_v)

        @pl.loop(0, G)
        def _(j):
            pltpu.sync_copy(msg_v.at[j], acc.at[dst_v.at[j]], add=True)

    plsc.subcore_barrier()
    _copy_out_stripe(acc, out_hbm, cid, sid)


@functools.partial(
    pl.kernel,
    out_type=jax.ShapeDtypeStruct((NC, NP), _f32),
    mesh=_mesh,
    scratch_types=[
        pltpu.VMEM((G, 128), jnp.int32),
        pltpu.VMEM((G, 128), jnp.int32),
        pltpu.VMEM((G, 128), _f32),
        pltpu.VMEM((STRIPE,), _f32),
        pltpu.VMEM((NP,), _f32),
        pltpu.VMEM_SHARED((NP,), _f32),
    ],
    compiler_params=_sc_params,
)
def _segsum_kernel(src_hbm, dst_hbm, val_hbm, out_hbm,
                   src_v, dst_v, msg_v, zbuf, val_v, acc):
    cid = lax.axis_index("c")
    sid = lax.axis_index("s")
    wid = cid * NS + sid

    _zero_acc_stripe(zbuf, acc, sid)
    pltpu.sync_copy(val_hbm, val_v)
    plsc.subcore_barrier()

    @pl.loop(0, BLKS)
    def _(blk):
        row0 = wid * ROWS_PER_W + blk * G
        pltpu.sync_copy(src_hbm.at[pl.ds(row0, G)], src_v)
        pltpu.sync_copy(dst_hbm.at[pl.ds(row0, G)], dst_v)

        @pl.loop(0, G)
        def _(j):
            for i in range(128 // L):
                idx = src_v[j, pl.ds(i * L, L)]
                msg_v[j, pl.ds(i * L, L)] = plsc.load_gather(val_v, [idx])
            pltpu.sync_copy(msg_v.at[j], acc.at[dst_v.at[j]], add=True)

    plsc.subcore_barrier()
    _copy_out_stripe(acc, out_hbm, cid, sid)


_R = NP // 128
_BR = 112
_GRID = _R // _BR

_vspec = pl.BlockSpec((_BR, 128), lambda i: (i, 0))


def _sspec():
    return pl.BlockSpec(memory_space=pltpu.SMEM)


def _node1_body(d0, d1, x, dinv, y):
    deg = d0[...] + d1[...] + 1.0
    inv = lax.rsqrt(jnp.maximum(deg, 1.0))
    dinv[...] = inv
    y[...] = inv * x[...]


_node1 = pl.pallas_call(
    _node1_body,
    grid=(_GRID,),
    in_specs=[_vspec, _vspec, _vspec],
    out_specs=[_vspec, _vspec],
    out_shape=[jax.ShapeDtypeStruct((_R, 128), _f32)] * 2,
)


def _node2_body(t0, t1, y, dinv, w1, b1, v, wout):
    s1 = dinv[...] * (t0[...] + t1[...] + y[...])
    acc = jnp.zeros_like(s1)
    for k in range(16):
        acc = acc + jnp.maximum(s1 * w1[0, k] + b1[0, k], 0.0) * v[0, k]
    wout[...] = dinv[...] * acc


_node2 = pl.pallas_call(
    _node2_body,
    grid=(_GRID,),
    in_specs=[_vspec, _vspec, _vspec, _vspec, _sspec(), _sspec(), _sspec()],
    out_specs=pl.BlockSpec((_BR, 128), lambda i: (i, 0)),
    out_shape=jax.ShapeDtypeStruct((_R, 128), _f32),
)


def _node3_body(t0, t1, w, dinv, c0, out):
    out[...] = dinv[...] * (t0[...] + t1[...] + w[...]) + c0[0]


_node3 = pl.pallas_call(
    _node3_body,
    grid=(_GRID,),
    in_specs=[_vspec, _vspec, _vspec, _vspec, _sspec()],
    out_specs=pl.BlockSpec((_BR, 128), lambda i: (i, 0)),
    out_shape=jax.ShapeDtypeStruct((_R, 128), _f32),
)


def kernel(x, edge_index, W1, b1, W2, b2, Wl, bl):
    E = edge_index.shape[1]
    xp = jnp.pad(x[:, 0], (0, NP - N))

    pad = jnp.full((EP - E,), N, jnp.int32)
    src2d = jnp.concatenate([edge_index[0], pad]).reshape(EP // 128, 128)
    dst2d = jnp.concatenate([edge_index[1], pad]).reshape(EP // 128, 128)

    degp = _deg_kernel(dst2d)
    dinv2d, y2d = _node1(degp[0].reshape(_R, 128), degp[1].reshape(_R, 128),
                         xp.reshape(_R, 128))

    t1p = _segsum_kernel(src2d, dst2d, y2d.reshape(NP))
    v = (W2 @ Wl).reshape(1, 16)
    w2d = _node2(t1p[0].reshape(_R, 128), t1p[1].reshape(_R, 128),
                 y2d, dinv2d, W1.reshape(1, 16), b1.reshape(1, 16), v)

    t2p = _segsum_kernel(src2d, dst2d, w2d.reshape(NP))
    c0 = (b2 @ Wl + bl).reshape(1)
    out2d = _node3(t2p[0].reshape(_R, 128), t2p[1].reshape(_R, 128),
                   w2d, dinv2d, c0)

    return out2d.reshape(NP)[:N].reshape(N, 1)


# speedup vs baseline: 168.9624x; 1.1004x over previous
import functools

import jax
import jax.numpy as jnp
from jax import lax
from jax.experimental import pallas as pl
from jax.experimental.pallas import tpu as pltpu
from jax.experimental.pallas import tpu_sc as plsc

N = 100000
NP = 100352
NC, NS, L = 2, 16, 16
NW = NC * NS
G = 32
ROWS_PER_W = 800
BLKS = ROWS_PER_W // G
EP = ROWS_PER_W * 128 * NW
STRIPE = NP // NS

_f32 = jnp.float32
_mesh = plsc.VectorSubcoreMesh(core_axis_name="c", subcore_axis_name="s")
_sc_params = pltpu.CompilerParams(needs_layout_passes=False)


def _zero_acc_stripe(zbuf, acc, sid):
    zeros = jnp.zeros((L,), _f32)

    @pl.loop(0, STRIPE // L)
    def _(i):
        zbuf[pl.ds(i * L, L)] = zeros

    pltpu.sync_copy(zbuf, acc.at[pl.ds(sid * STRIPE, STRIPE)])


def _copy_out_stripe(acc, out_hbm, cid, sid):
    pltpu.sync_copy(
        acc.at[pl.ds(sid * STRIPE, STRIPE)],
        out_hbm.at[cid, pl.ds(sid * STRIPE, STRIPE)],
    )


@functools.partial(
    pl.kernel,
    out_type=jax.ShapeDtypeStruct((NC, NP), _f32),
    mesh=_mesh,
    scratch_types=[
        pltpu.VMEM((G, 128), jnp.int32),
        pltpu.VMEM((1, 128), _f32),
        pltpu.VMEM((STRIPE,), _f32),
        pltpu.VMEM_SHARED((NP,), _f32),
        pltpu.SemaphoreType.DMA,
    ],
    compiler_params=_sc_params,
)
def _deg_kernel(dst_hbm, out_hbm, dst_v, ones_v, zbuf, acc, sem):
    cid = lax.axis_index("c")
    sid = lax.axis_index("s")
    wid = cid * NS + sid

    ones = jnp.ones((L,), _f32)
    for i in range(128 // L):
        ones_v[0, pl.ds(i * L, L)] = ones

    _zero_acc_stripe(zbuf, acc, sid)
    plsc.subcore_barrier()

    @pl.loop(0, BLKS)
    def _(blk):
        row0 = wid * ROWS_PER_W + blk * G
        pltpu.sync_copy(dst_hbm.at[pl.ds(row0, G)], dst_v)

        @pl.loop(0, G)
        def _(j):
            pltpu.async_copy(ones_v.at[0], acc.at[dst_v.at[j]], sem, add=True)

        @pl.loop(0, G)
        def _(j):
            pltpu.make_async_copy(ones_v.at[0], acc.at[dst_v.at[j]], sem).wait()

    plsc.subcore_barrier()
    _copy_out_stripe(acc, out_hbm, cid, sid)


@functools.partial(
    pl.kernel,
    out_type=jax.ShapeDtypeStruct((NC, NP), _f32),
    mesh=_mesh,
    scratch_types=[
        pltpu.VMEM((G, 128), jnp.int32),
        pltpu.VMEM((G, 128), jnp.int32),
        pltpu.VMEM((G, 128), _f32),
        pltpu.VMEM((STRIPE,), _f32),
        pltpu.VMEM((NP,), _f32),
        pltpu.VMEM_SHARED((NP,), _f32),
        pltpu.SemaphoreType.DMA,
    ],
    compiler_params=_sc_params,
)
def _segsum_kernel(src_hbm, dst_hbm, val_hbm, out_hbm,
                   src_v, dst_v, msg_v, zbuf, val_v, acc, sem):
    cid = lax.axis_index("c")
    sid = lax.axis_index("s")
    wid = cid * NS + sid

    pltpu.sync_copy(val_hbm, val_v)
    _zero_acc_stripe(zbuf, acc, sid)
    plsc.subcore_barrier()

    @pl.loop(0, BLKS)
    def _(blk):
        row0 = wid * ROWS_PER_W + blk * G
        pltpu.sync_copy(src_hbm.at[pl.ds(row0, G)], src_v)
        pltpu.sync_copy(dst_hbm.at[pl.ds(row0, G)], dst_v)

        @pl.loop(0, G)
        def _(j):
            for i in range(128 // L):
                idx = src_v[j, pl.ds(i * L, L)]
                msg_v[j, pl.ds(i * L, L)] = plsc.load_gather(val_v, [idx])
            pltpu.async_copy(msg_v.at[j], acc.at[dst_v.at[j]], sem, add=True)

        @pl.loop(0, G)
        def _(j):
            pltpu.make_async_copy(msg_v.at[j], acc.at[dst_v.at[j]], sem).wait()

    plsc.subcore_barrier()
    _copy_out_stripe(acc, out_hbm, cid, sid)


_R = NP // 128
_BR = 112
_GRID = _R // _BR

_vspec = pl.BlockSpec((_BR, 128), lambda i: (i, 0))


def _sspec():
    return pl.BlockSpec(memory_space=pltpu.SMEM)


def _node1_body(d0, d1, x, dinv, y):
    deg = d0[...] + d1[...] + 1.0
    inv = lax.rsqrt(jnp.maximum(deg, 1.0))
    dinv[...] = inv
    y[...] = inv * x[...]


_node1 = pl.pallas_call(
    _node1_body,
    grid=(_GRID,),
    in_specs=[_vspec, _vspec, _vspec],
    out_specs=[_vspec, _vspec],
    out_shape=[jax.ShapeDtypeStruct((_R, 128), _f32)] * 2,
)


def _node2_body(t0, t1, y, dinv, w1, b1, v, wout):
    s1 = dinv[...] * (t0[...] + t1[...] + y[...])
    acc = jnp.zeros_like(s1)
    for k in range(16):
        acc = acc + jnp.maximum(s1 * w1[0, k] + b1[0, k], 0.0) * v[0, k]
    wout[...] = dinv[...] * acc


_node2 = pl.pallas_call(
    _node2_body,
    grid=(_GRID,),
    in_specs=[_vspec, _vspec, _vspec, _vspec, _sspec(), _sspec(), _sspec()],
    out_specs=pl.BlockSpec((_BR, 128), lambda i: (i, 0)),
    out_shape=jax.ShapeDtypeStruct((_R, 128), _f32),
)


def _node3_body(t0, t1, w, dinv, c0, out):
    out[...] = dinv[...] * (t0[...] + t1[...] + w[...]) + c0[0]


_node3 = pl.pallas_call(
    _node3_body,
    grid=(_GRID,),
    in_specs=[_vspec, _vspec, _vspec, _vspec, _sspec()],
    out_specs=pl.BlockSpec((_BR, 128), lambda i: (i, 0)),
    out_shape=jax.ShapeDtypeStruct((_R, 128), _f32),
)


def kernel(x, edge_index, W1, b1, W2, b2, Wl, bl):
    E = edge_index.shape[1]
    xp = jnp.pad(x[:, 0], (0, NP - N))

    pad = jnp.full((EP - E,), N, jnp.int32)
    src2d = jnp.concatenate([edge_index[0], pad]).reshape(EP // 128, 128)
    dst2d = jnp.concatenate([edge_index[1], pad]).reshape(EP // 128, 128)

    degp = _deg_kernel(dst2d)
    dinv2d, y2d = _node1(degp[0].reshape(_R, 128), degp[1].reshape(_R, 128),
                         xp.reshape(_R, 128))

    t1p = _segsum_kernel(src2d, dst2d, y2d.reshape(NP))
    v = (W2 @ Wl).reshape(1, 16)
    w2d = _node2(t1p[0].reshape(_R, 128), t1p[1].reshape(_R, 128),
                 y2d, dinv2d, W1.reshape(1, 16), b1.reshape(1, 16), v)

    t2p = _segsum_kernel(src2d, dst2d, w2d.reshape(NP))
    c0 = (b2 @ Wl + bl).reshape(1)
    out2d = _node3(t2p[0].reshape(_R, 128), t2p[1].reshape(_R, 128),
                   w2d, dinv2d, c0)

    return out2d.reshape(NP)[:N].reshape(N, 1)


# speedup vs baseline: 224.5160x; 1.3288x over previous
import functools

import jax
import jax.numpy as jnp
from jax import lax
from jax.experimental import pallas as pl
from jax.experimental.pallas import tpu as pltpu
from jax.experimental.pallas import tpu_sc as plsc

N = 100000
NP = 100352
NC, NS, L = 2, 16, 16
E = 3200000
ROWS = E // 128
G = 24
R0_BASE, R0_REM = 984, 8
R1_BASE = (ROWS - (R0_BASE * NS + R0_REM)) // NS
STRIPE = NP // NS

_f32 = jnp.float32
_mesh = plsc.VectorSubcoreMesh(core_axis_name="c", subcore_axis_name="s")
_sc_params = pltpu.CompilerParams(needs_layout_passes=False,
                                  use_tc_tiling_on_sc=False)


def _worker_rows(cid, sid):
    c0_total = R0_BASE * NS + R0_REM
    row0_c0 = sid * R0_BASE + jnp.minimum(sid, R0_REM)
    nrows_c0 = R0_BASE + (sid < R0_REM).astype(jnp.int32)
    row0_c1 = c0_total + sid * R1_BASE
    nrows_c1 = jnp.int32(R1_BASE)
    is0 = cid == 0
    row0 = jnp.where(is0, row0_c0, row0_c1)
    nrows = jnp.where(is0, nrows_c0, nrows_c1)
    return row0, nrows


def _zero_acc_stripe(zbuf, acc, sid):
    zeros = jnp.zeros((L,), _f32)

    @pl.loop(0, STRIPE // L)
    def _(i):
        zbuf[pl.ds(i * L, L)] = zeros

    pltpu.sync_copy(zbuf, acc.at[pl.ds(sid * STRIPE, STRIPE)])


def _copy_out_stripe(acc, out_hbm, cid, sid):
    pltpu.sync_copy(
        acc.at[pl.ds(sid * STRIPE, STRIPE)],
        out_hbm.at[cid, pl.ds(sid * STRIPE, STRIPE)],
    )


@functools.partial(
    pl.kernel,
    out_type=jax.ShapeDtypeStruct((NC, NP), _f32),
    mesh=_mesh,
    scratch_types=[
        pltpu.VMEM((G, 128), jnp.int32),
        pltpu.VMEM((1, 128), _f32),
        pltpu.VMEM((STRIPE,), _f32),
        pltpu.VMEM_SHARED((NP,), _f32),
        pltpu.SemaphoreType.DMA,
    ],
    compiler_params=_sc_params,
)
def _deg_kernel(ei_hbm, out_hbm, dst_v, ones_v, zbuf, acc, sem):
    cid = lax.axis_index("c")
    sid = lax.axis_index("s")
    row0, nrows = _worker_rows(cid, sid)

    ones = jnp.ones((L,), _f32)
    for i in range(128 // L):
        ones_v[0, pl.ds(i * L, L)] = ones

    _zero_acc_stripe(zbuf, acc, sid)
    plsc.subcore_barrier()

    full = nrows // G

    @pl.loop(0, full)
    def _(blk):
        r = row0 + blk * G
        pltpu.sync_copy(ei_hbm.at[1, pl.ds(r, G)], dst_v)

        @pl.loop(0, G)
        def _(j):
            pltpu.async_copy(ones_v.at[0], acc.at[dst_v.at[j]], sem, add=True)

        @pl.loop(0, G)
        def _(j):
            pltpu.make_async_copy(ones_v.at[0], acc.at[dst_v.at[j]], sem).wait()

    @pl.loop(full * G, nrows)
    def _(t):
        pltpu.sync_copy(ei_hbm.at[1, pl.ds(row0 + t, 1)], dst_v.at[pl.ds(0, 1)])
        pltpu.sync_copy(ones_v.at[0], acc.at[dst_v.at[0]], add=True)

    plsc.subcore_barrier()
    _copy_out_stripe(acc, out_hbm, cid, sid)


@functools.partial(
    pl.kernel,
    out_type=jax.ShapeDtypeStruct((NC, NP), _f32),
    mesh=_mesh,
    scratch_types=[
        pltpu.VMEM((G, 128), jnp.int32),
        pltpu.VMEM((G, 128), jnp.int32),
        pltpu.VMEM((G, 128), _f32),
        pltpu.VMEM((STRIPE,), _f32),
        pltpu.VMEM((NP,), _f32),
        pltpu.VMEM_SHARED((NP,), _f32),
        pltpu.SemaphoreType.DMA,
        pltpu.SemaphoreType.DMA,
    ],
    compiler_params=_sc_params,
)
def _segsum_kernel(ei_hbm, val_hbm, out_hbm,
                   src_v, dst_v, msg_v, zbuf, val_v, acc, lsem, sem):
    cid = lax.axis_index("c")
    sid = lax.axis_index("s")
    row0, nrows = _worker_rows(cid, sid)

    load = pltpu.async_copy(val_hbm, val_v, lsem)
    _zero_acc_stripe(zbuf, acc, sid)
    load.wait()
    plsc.subcore_barrier()

    full = nrows // G

    @pl.loop(0, full)
    def _(blk):
        r = row0 + blk * G
        pltpu.sync_copy(ei_hbm.at[0, pl.ds(r, G)], src_v)
        pltpu.sync_copy(ei_hbm.at[1, pl.ds(r, G)], dst_v)

        @pl.loop(0, G)
        def _(j):
            for i in range(128 // L):
                idx = src_v[j, pl.ds(i * L, L)]
                msg_v[j, pl.ds(i * L, L)] = plsc.load_gather(val_v, [idx])
            pltpu.async_copy(msg_v.at[j], acc.at[dst_v.at[j]], sem, add=True)

        @pl.loop(0, G)
        def _(j):
            pltpu.make_async_copy(msg_v.at[j], acc.at[dst_v.at[j]], sem).wait()

    @pl.loop(full * G, nrows)
    def _(t):
        r = row0 + t
        pltpu.sync_copy(ei_hbm.at[0, pl.ds(r, 1)], src_v.at[pl.ds(0, 1)])
        pltpu.sync_copy(ei_hbm.at[1, pl.ds(r, 1)], dst_v.at[pl.ds(0, 1)])
        for i in range(128 // L):
            idx = src_v[0, pl.ds(i * L, L)]
            msg_v[0, pl.ds(i * L, L)] = plsc.load_gather(val_v, [idx])
        pltpu.sync_copy(msg_v.at[0], acc.at[dst_v.at[0]], add=True)

    plsc.subcore_barrier()
    _copy_out_stripe(acc, out_hbm, cid, sid)


_R = NP // 128
_BR = 112
_GRID = _R // _BR

_vspec = pl.BlockSpec((_BR, 128), lambda i: (i, 0))


def _sspec():
    return pl.BlockSpec(memory_space=pltpu.SMEM)


def _node1_body(d0, d1, x, dinv, y):
    deg = d0[...] + d1[...] + 1.0
    inv = lax.rsqrt(jnp.maximum(deg, 1.0))
    dinv[...] = inv
    y[...] = inv * x[...]


_node1 = pl.pallas_call(
    _node1_body,
    grid=(_GRID,),
    in_specs=[_vspec, _vspec, _vspec],
    out_specs=[_vspec, _vspec],
    out_shape=[jax.ShapeDtypeStruct((_R, 128), _f32)] * 2,
)


def _node2_body(t0, t1, y, dinv, w1, b1, v, wout):
    s1 = dinv[...] * (t0[...] + t1[...] + y[...])
    acc = jnp.zeros_like(s1)
    for k in range(16):
        acc = acc + jnp.maximum(s1 * w1[0, k] + b1[0, k], 0.0) * v[0, k]
    wout[...] = dinv[...] * acc


_node2 = pl.pallas_call(
    _node2_body,
    grid=(_GRID,),
    in_specs=[_vspec, _vspec, _vspec, _vspec, _sspec(), _sspec(), _sspec()],
    out_specs=pl.BlockSpec((_BR, 128), lambda i: (i, 0)),
    out_shape=jax.ShapeDtypeStruct((_R, 128), _f32),
)


def _node3_body(t0, t1, w, dinv, c0, out):
    out[...] = dinv[...] * (t0[...] + t1[...] + w[...]) + c0[0]


_node3 = pl.pallas_call(
    _node3_body,
    grid=(_GRID,),
    in_specs=[_vspec, _vspec, _vspec, _vspec, _sspec()],
    out_specs=pl.BlockSpec((_BR, 128), lambda i: (i, 0)),
    out_shape=jax.ShapeDtypeStruct((_R, 128), _f32),
)


def kernel(x, edge_index, W1, b1, W2, b2, Wl, bl):
    assert edge_index.shape == (2, E)
    ei3 = edge_index.reshape(2, ROWS, 128)
    xp = jnp.pad(x[:, 0], (0, NP - N))

    degp = _deg_kernel(ei3)
    dinv2d, y2d = _node1(degp[0].reshape(_R, 128), degp[1].reshape(_R, 128),
                         xp.reshape(_R, 128))

    t1p = _segsum_kernel(ei3, y2d.reshape(NP))
    v = (W2 @ Wl).reshape(1, 16)
    w2d = _node2(t1p[0].reshape(_R, 128), t1p[1].reshape(_R, 128),
                 y2d, dinv2d, W1.reshape(1, 16), b1.reshape(1, 16), v)

    t2p = _segsum_kernel(ei3, w2d.reshape(NP))
    c0 = (b2 @ Wl + bl).reshape(1)
    out2d = _node3(t2p[0].reshape(_R, 128), t2p[1].reshape(_R, 128),
                   w2d, dinv2d, c0)

    return out2d.reshape(NP)[:N].reshape(N, 1)


# speedup vs baseline: 257.9342x; 1.1488x over previous
import functools

import jax
import jax.numpy as jnp
from jax import lax
from jax.experimental import pallas as pl
from jax.experimental.pallas import tpu as pltpu
from jax.experimental.pallas import tpu_sc as plsc

N = 100000
NP = 100352
NC, NS, L = 2, 16, 16
E = 3200000
ROWS = E // 128
G = 8
UNITS = ROWS // G
U0_BASE, U0_REM = 113, 5
U1_BASE = (UNITS - (U0_BASE * NS + U0_REM)) // NS
STRIPE = NP // NS

_f32 = jnp.float32
_mesh = plsc.VectorSubcoreMesh(core_axis_name="c", subcore_axis_name="s")
_sc_params = pltpu.CompilerParams(needs_layout_passes=False,
                                  use_tc_tiling_on_sc=False)


def _worker_units(cid, sid):
    c0_total = U0_BASE * NS + U0_REM
    u0_c0 = sid * U0_BASE + jnp.minimum(sid, U0_REM)
    n_c0 = U0_BASE + (sid < U0_REM).astype(jnp.int32)
    u0_c1 = c0_total + sid * U1_BASE
    n_c1 = jnp.int32(U1_BASE)
    is0 = cid == 0
    u0 = jnp.where(is0, u0_c0, u0_c1)
    nunits = jnp.where(is0, n_c0, n_c1)
    return u0, nunits


def _zero_acc_stripe(zbuf, acc, sid):
    zeros = jnp.zeros((L,), _f32)

    @pl.loop(0, STRIPE // L)
    def _(i):
        zbuf[pl.ds(i * L, L)] = zeros

    pltpu.sync_copy(zbuf, acc.at[pl.ds(sid * STRIPE, STRIPE)])


def _copy_out_stripe(acc, out_hbm, cid, sid):
    pltpu.sync_copy(
        acc.at[pl.ds(sid * STRIPE, STRIPE)],
        out_hbm.at[cid, pl.ds(sid * STRIPE, STRIPE)],
    )


@functools.partial(
    pl.kernel,
    out_type=jax.ShapeDtypeStruct((NC, NP), _f32),
    mesh=_mesh,
    scratch_types=[
        pltpu.VMEM((2, G, 128), jnp.int32),
        pltpu.VMEM((1, 128), _f32),
        pltpu.VMEM((STRIPE,), _f32),
        pltpu.VMEM_SHARED((NP,), _f32),
        pltpu.SemaphoreType.DMA,
        pltpu.SemaphoreType.DMA,
    ],
    compiler_params=_sc_params,
)
def _deg_kernel(ei_hbm, out_hbm, dst_v, ones_v, zbuf, acc, ssem, sem):
    cid = lax.axis_index("c")
    sid = lax.axis_index("s")
    u0, nunits = _worker_units(cid, sid)

    def fire_stage(u, p):
        pltpu.async_copy(ei_hbm.at[1, pl.ds((u0 + u) * G, G)], dst_v.at[p],
                         ssem)

    def wait_stage(p):
        pltpu.make_async_copy(ei_hbm.at[1, pl.ds(0, G)], dst_v.at[p],
                              ssem).wait()

    fire_stage(0, 0)

    ones = jnp.ones((L,), _f32)
    for i in range(128 // L):
        ones_v[0, pl.ds(i * L, L)] = ones

    _zero_acc_stripe(zbuf, acc, sid)
    plsc.subcore_barrier()

    @pl.loop(0, nunits)
    def _(blk):
        p = blk % 2
        wait_stage(p)

        @pl.when(blk + 1 < nunits)
        def _():
            fire_stage(blk + 1, 1 - p)

        for j in range(G):
            pltpu.async_copy(ones_v.at[0], acc.at[dst_v.at[p, j]], sem,
                             add=True)
        for j in range(G):
            pltpu.make_async_copy(ones_v.at[0], acc.at[dst_v.at[p, j]],
                                  sem).wait()

    plsc.subcore_barrier()
    _copy_out_stripe(acc, out_hbm, cid, sid)


@functools.partial(
    pl.kernel,
    out_type=jax.ShapeDtypeStruct((NC, NP), _f32),
    mesh=_mesh,
    scratch_types=[
        pltpu.VMEM((2, G, 128), jnp.int32),
        pltpu.VMEM((2, G, 128), jnp.int32),
        pltpu.VMEM((G, 128), _f32),
        pltpu.VMEM((STRIPE,), _f32),
        pltpu.VMEM((NP,), _f32),
        pltpu.VMEM_SHARED((NP,), _f32),
        pltpu.SemaphoreType.DMA,
        pltpu.SemaphoreType.DMA,
        pltpu.SemaphoreType.DMA,
    ],
    compiler_params=_sc_params,
)
def _segsum_kernel(ei_hbm, val_hbm, out_hbm,
                   src_v, dst_v, msg_v, zbuf, val_v, acc, lsem, ssem, sem):
    cid = lax.axis_index("c")
    sid = lax.axis_index("s")
    u0, nunits = _worker_units(cid, sid)

    def fire_stage(u, p):
        r = (u0 + u) * G
        pltpu.async_copy(ei_hbm.at[0, pl.ds(r, G)], src_v.at[p], ssem)
        pltpu.async_copy(ei_hbm.at[1, pl.ds(r, G)], dst_v.at[p], ssem)

    def wait_stage(p):
        pltpu.make_async_copy(ei_hbm.at[0, pl.ds(0, G)], src_v.at[p],
                              ssem).wait()
        pltpu.make_async_copy(ei_hbm.at[1, pl.ds(0, G)], dst_v.at[p],
                              ssem).wait()

    fire_stage(0, 0)
    load = pltpu.async_copy(val_hbm, val_v, lsem)
    _zero_acc_stripe(zbuf, acc, sid)
    load.wait()
    plsc.subcore_barrier()

    @pl.loop(0, nunits)
    def _(blk):
        p = blk % 2
        wait_stage(p)

        @pl.when(blk + 1 < nunits)
        def _():
            fire_stage(blk + 1, 1 - p)

        for j in range(G):
            for i in range(128 // L):
                idx = src_v[p, j, pl.ds(i * L, L)]
                msg_v[j, pl.ds(i * L, L)] = plsc.load_gather(val_v, [idx])
            pltpu.async_copy(msg_v.at[j], acc.at[dst_v.at[p, j]], sem,
                             add=True)
        for j in range(G):
            pltpu.make_async_copy(msg_v.at[j], acc.at[dst_v.at[p, j]],
                                  sem).wait()

    plsc.subcore_barrier()
    _copy_out_stripe(acc, out_hbm, cid, sid)


_R = NP // 128
_BR = 112
_GRID = _R // _BR

_vspec = pl.BlockSpec((_BR, 128), lambda i: (i, 0))


def _sspec():
    return pl.BlockSpec(memory_space=pltpu.SMEM)


def _node1_body(d0, d1, x, dinv, y):
    deg = d0[...] + d1[...] + 1.0
    inv = lax.rsqrt(jnp.maximum(deg, 1.0))
    dinv[...] = inv
    y[...] = inv * x[...]


_node1 = pl.pallas_call(
    _node1_body,
    grid=(_GRID,),
    in_specs=[_vspec, _vspec, _vspec],
    out_specs=[_vspec, _vspec],
    out_shape=[jax.ShapeDtypeStruct((_R, 128), _f32)] * 2,
)


def _node2_body(t0, t1, y, dinv, w1, b1, v, wout):
    s1 = dinv[...] * (t0[...] + t1[...] + y[...])
    acc = jnp.zeros_like(s1)
    for k in range(16):
        acc = acc + jnp.maximum(s1 * w1[0, k] + b1[0, k], 0.0) * v[0, k]
    wout[...] = dinv[...] * acc


_node2 = pl.pallas_call(
    _node2_body,
    grid=(_GRID,),
    in_specs=[_vspec, _vspec, _vspec, _vspec, _sspec(), _sspec(), _sspec()],
    out_specs=pl.BlockSpec((_BR, 128), lambda i: (i, 0)),
    out_shape=jax.ShapeDtypeStruct((_R, 128), _f32),
)


def _node3_body(t0, t1, w, dinv, c0, out):
    out[...] = dinv[...] * (t0[...] + t1[...] + w[...]) + c0[0]


_node3 = pl.pallas_call(
    _node3_body,
    grid=(_GRID,),
    in_specs=[_vspec, _vspec, _vspec, _vspec, _sspec()],
    out_specs=pl.BlockSpec((_BR, 128), lambda i: (i, 0)),
    out_shape=jax.ShapeDtypeStruct((_R, 128), _f32),
)


def kernel(x, edge_index, W1, b1, W2, b2, Wl, bl):
    assert edge_index.shape == (2, E)
    ei3 = edge_index.reshape(2, ROWS, 128)
    xp = jnp.pad(x[:, 0], (0, NP - N))

    degp = _deg_kernel(ei3)
    dinv2d, y2d = _node1(degp[0].reshape(_R, 128), degp[1].reshape(_R, 128),
                         xp.reshape(_R, 128))

    t1p = _segsum_kernel(ei3, y2d.reshape(NP))
    v = (W2 @ Wl).reshape(1, 16)
    w2d = _node2(t1p[0].reshape(_R, 128), t1p[1].reshape(_R, 128),
                 y2d, dinv2d, W1.reshape(1, 16), b1.reshape(1, 16), v)

    t2p = _segsum_kernel(ei3, w2d.reshape(NP))
    c0 = (b2 @ Wl + bl).reshape(1)
    out2d = _node3(t2p[0].reshape(_R, 128), t2p[1].reshape(_R, 128),
                   w2d, dinv2d, c0)

    return out2d.reshape(NP)[:N].reshape(N, 1)


# speedup vs baseline: 262.3383x; 1.0171x over previous
import functools

import jax
import jax.numpy as jnp
from jax import lax
from jax.experimental import pallas as pl
from jax.experimental.pallas import tpu as pltpu
from jax.experimental.pallas import tpu_sc as plsc

N = 100000
NP = 100352
NC, NS, L = 2, 16, 16
E = 3200000
ROWS = E // 128
G = 8
UNITS = ROWS // G
U0_BASE, U0_REM = 108, 6
U1_TOTAL = UNITS - (U0_BASE * NS + U0_REM)
U1_BASE, U1_REM = U1_TOTAL // NS, U1_TOTAL % NS
STRIPE = NP // NS

_f32 = jnp.float32
_mesh = plsc.VectorSubcoreMesh(core_axis_name="c", subcore_axis_name="s")
_sc_params = pltpu.CompilerParams(needs_layout_passes=False)


def _worker_units(cid, sid):
    c0_total = U0_BASE * NS + U0_REM
    u0_c0 = sid * U0_BASE + jnp.minimum(sid, U0_REM)
    n_c0 = U0_BASE + (sid < U0_REM).astype(jnp.int32)
    u0_c1 = c0_total + sid * U1_BASE + jnp.minimum(sid, U1_REM)
    n_c1 = U1_BASE + (sid < U1_REM).astype(jnp.int32)
    is0 = cid == 0
    u0 = jnp.where(is0, u0_c0, u0_c1)
    nunits = jnp.where(is0, n_c0, n_c1)
    return u0, nunits


def _zero_acc_stripe(zbuf, acc, sid):
    zeros = jnp.zeros((L,), _f32)

    @pl.loop(0, STRIPE // L)
    def _(i):
        zbuf[pl.ds(i * L, L)] = zeros

    pltpu.sync_copy(zbuf, acc.at[pl.ds(sid * STRIPE, STRIPE)])


def _copy_out_stripe(acc, out_hbm, cid, sid):
    pltpu.sync_copy(
        acc.at[pl.ds(sid * STRIPE, STRIPE)],
        out_hbm.at[cid, pl.ds(sid * STRIPE, STRIPE)],
    )


@functools.partial(
    pl.kernel,
    out_type=jax.ShapeDtypeStruct((NC, NP), _f32),
    mesh=_mesh,
    scratch_types=[
        pltpu.VMEM((2, G, 128), jnp.int32),
        pltpu.VMEM((1, 128), _f32),
        pltpu.VMEM((STRIPE,), _f32),
        pltpu.VMEM_SHARED((NP,), _f32),
        pltpu.SemaphoreType.DMA,
        pltpu.SemaphoreType.DMA,
    ],
    compiler_params=_sc_params,
)
def _deg_kernel(ei_hbm, out_hbm, dst_v, ones_v, zbuf, acc, ssem, sem):
    cid = lax.axis_index("c")
    sid = lax.axis_index("s")
    u0, nunits = _worker_units(cid, sid)

    def fire_stage(u, p):
        pltpu.async_copy(ei_hbm.at[1, pl.ds((u0 + u) * G, G)], dst_v.at[p],
                         ssem)

    def wait_stage(p):
        pltpu.make_async_copy(ei_hbm.at[1, pl.ds(0, G)], dst_v.at[p],
                              ssem).wait()

    fire_stage(0, 0)

    ones = jnp.ones((L,), _f32)
    for i in range(128 // L):
        ones_v[0, pl.ds(i * L, L)] = ones

    _zero_acc_stripe(zbuf, acc, sid)
    plsc.subcore_barrier()

    def drain(p):
        for j in range(G):
            pltpu.make_async_copy(ones_v.at[0], acc.at[dst_v.at[p, j]],
                                  sem).wait()

    @pl.loop(0, nunits)
    def _(blk):
        p = blk % 2
        wait_stage(p)

        @pl.when(blk > 0)
        def _():
            drain(1 - p)

        @pl.when(blk + 1 < nunits)
        def _():
            fire_stage(blk + 1, 1 - p)

        for j in range(G):
            pltpu.async_copy(ones_v.at[0], acc.at[dst_v.at[p, j]], sem,
                             add=True)

    drain((nunits - 1) % 2)
    plsc.subcore_barrier()
    _copy_out_stripe(acc, out_hbm, cid, sid)


@functools.partial(
    pl.kernel,
    out_type=jax.ShapeDtypeStruct((NC, NP), _f32),
    mesh=_mesh,
    scratch_types=[
        pltpu.VMEM((2, G, 128), jnp.int32),
        pltpu.VMEM((2, G, 128), jnp.int32),
        pltpu.VMEM((2, G, 128), _f32),
        pltpu.VMEM((STRIPE,), _f32),
        pltpu.VMEM((NP,), _f32),
        pltpu.VMEM_SHARED((NP,), _f32),
        pltpu.SemaphoreType.DMA,
        pltpu.SemaphoreType.DMA,
        pltpu.SemaphoreType.DMA,
    ],
    compiler_params=_sc_params,
)
def _segsum_kernel(ei_hbm, val_hbm, out_hbm,
                   src_v, dst_v, msg_v, zbuf, val_v, acc, lsem, ssem, sem):
    cid = lax.axis_index("c")
    sid = lax.axis_index("s")
    u0, nunits = _worker_units(cid, sid)

    def fire_stage(u, p):
        r = (u0 + u) * G
        pltpu.async_copy(ei_hbm.at[0, pl.ds(r, G)], src_v.at[p], ssem)
        pltpu.async_copy(ei_hbm.at[1, pl.ds(r, G)], dst_v.at[p], ssem)

    def wait_stage(p):
        pltpu.make_async_copy(ei_hbm.at[0, pl.ds(0, G)], src_v.at[p],
                              ssem).wait()
        pltpu.make_async_copy(ei_hbm.at[1, pl.ds(0, G)], dst_v.at[p],
                              ssem).wait()

    fire_stage(0, 0)
    load = pltpu.async_copy(val_hbm, val_v, lsem)
    _zero_acc_stripe(zbuf, acc, sid)
    load.wait()
    plsc.subcore_barrier()

    def drain(p):
        for j in range(G):
            pltpu.make_async_copy(msg_v.at[p, j], acc.at[dst_v.at[p, j]],
                                  sem).wait()

    @pl.loop(0, nunits)
    def _(blk):
        p = blk % 2
        wait_stage(p)

        @pl.when(blk > 0)
        def _():
            drain(1 - p)

        @pl.when(blk + 1 < nunits)
        def _():
            fire_stage(blk + 1, 1 - p)

        for j in range(G):
            for i in range(128 // L):
                idx = src_v[p, j, pl.ds(i * L, L)]
                msg_v[p, j, pl.ds(i * L, L)] = plsc.load_gather(val_v, [idx])
            pltpu.async_copy(msg_v.at[p, j], acc.at[dst_v.at[p, j]], sem,
                             add=True)

    drain((nunits - 1) % 2)
    plsc.subcore_barrier()
    _copy_out_stripe(acc, out_hbm, cid, sid)


_R = NP // 128
_BR = 112
_GRID = _R // _BR

_vspec = pl.BlockSpec((_BR, 128), lambda i: (i, 0))


def _sspec():
    return pl.BlockSpec(memory_space=pltpu.SMEM)


def _node1_body(d0, d1, x, dinv, y):
    deg = d0[...] + d1[...] + 1.0
    inv = lax.rsqrt(jnp.maximum(deg, 1.0))
    dinv[...] = inv
    y[...] = inv * x[...]


_node1 = pl.pallas_call(
    _node1_body,
    grid=(_GRID,),
    in_specs=[_vspec, _vspec, _vspec],
    out_specs=[_vspec, _vspec],
    out_shape=[jax.ShapeDtypeStruct((_R, 128), _f32)] * 2,
)


def _node2_body(t0, t1, y, dinv, w1, b1, v, wout):
    s1 = dinv[...] * (t0[...] + t1[...] + y[...])
    acc = jnp.zeros_like(s1)
    for k in range(16):
        acc = acc + jnp.maximum(s1 * w1[0, k] + b1[0, k], 0.0) * v[0, k]
    wout[...] = dinv[...] * acc


_node2 = pl.pallas_call(
    _node2_body,
    grid=(_GRID,),
    in_specs=[_vspec, _vspec, _vspec, _vspec, _sspec(), _sspec(), _sspec()],
    out_specs=pl.BlockSpec((_BR, 128), lambda i: (i, 0)),
    out_shape=jax.ShapeDtypeStruct((_R, 128), _f32),
)


def _node3_body(t0, t1, w, dinv, c0, out):
    out[...] = dinv[...] * (t0[...] + t1[...] + w[...]) + c0[0]


_node3 = pl.pallas_call(
    _node3_body,
    grid=(_GRID,),
    in_specs=[_vspec, _vspec, _vspec, _vspec, _sspec()],
    out_specs=pl.BlockSpec((_BR, 128), lambda i: (i, 0)),
    out_shape=jax.ShapeDtypeStruct((_R, 128), _f32),
)


def kernel(x, edge_index, W1, b1, W2, b2, Wl, bl):
    assert edge_index.shape == (2, E)
    ei3 = edge_index.reshape(2, ROWS, 128)
    xp = jnp.pad(x[:, 0], (0, NP - N))

    degp = _deg_kernel(ei3)
    dinv2d, y2d = _node1(degp[0].reshape(_R, 128), degp[1].reshape(_R, 128),
                         xp.reshape(_R, 128))

    t1p = _segsum_kernel(ei3, y2d.reshape(NP))
    v = (W2 @ Wl).reshape(1, 16)
    w2d = _node2(t1p[0].reshape(_R, 128), t1p[1].reshape(_R, 128),
                 y2d, dinv2d, W1.reshape(1, 16), b1.reshape(1, 16), v)

    t2p = _segsum_kernel(ei3, w2d.reshape(NP))
    c0 = (b2 @ Wl + bl).reshape(1)
    out2d = _node3(t2p[0].reshape(_R, 128), t2p[1].reshape(_R, 128),
                   w2d, dinv2d, c0)

    return out2d.reshape(NP)[:N].reshape(N, 1)


# speedup vs baseline: 281.5350x; 1.0732x over previous
import functools

import jax
import jax.numpy as jnp
from jax import lax
from jax.experimental import pallas as pl
from jax.experimental.pallas import tpu as pltpu
from jax.experimental.pallas import tpu_sc as plsc

N = 100000
NP = 100352
NC, NS, L = 2, 16, 16
E = 3200000
ROWS = E // 128
G = 8
UNITS = ROWS // G
U0_BASE, U0_REM = 97, 11
U1_TOTAL = UNITS - (U0_BASE * NS + U0_REM)
U1_BASE, U1_REM = U1_TOTAL // NS, U1_TOTAL % NS
STRIPE = NP // NS

_f32 = jnp.float32
_mesh = plsc.VectorSubcoreMesh(core_axis_name="c", subcore_axis_name="s")
_sc_params = pltpu.CompilerParams(needs_layout_passes=False)


def _worker_units(cid, sid):
    c0_total = U0_BASE * NS + U0_REM
    u0_c0 = sid * U0_BASE + jnp.minimum(sid, U0_REM)
    n_c0 = U0_BASE + (sid < U0_REM).astype(jnp.int32)
    u0_c1 = c0_total + sid * U1_BASE + jnp.minimum(sid, U1_REM)
    n_c1 = U1_BASE + (sid < U1_REM).astype(jnp.int32)
    is0 = cid == 0
    u0 = jnp.where(is0, u0_c0, u0_c1)
    nunits = jnp.where(is0, n_c0, n_c1)
    return u0, nunits


def _zero_acc_stripe(zbuf, acc, sid):
    zeros = jnp.zeros((L,), _f32)

    @pl.loop(0, STRIPE // L)
    def _(i):
        zbuf[pl.ds(i * L, L)] = zeros

    pltpu.sync_copy(zbuf, acc.at[pl.ds(sid * STRIPE, STRIPE)])


def _copy_out_stripe(acc, out_hbm, cid, sid):
    pltpu.sync_copy(
        acc.at[pl.ds(sid * STRIPE, STRIPE)],
        out_hbm.at[cid, pl.ds(sid * STRIPE, STRIPE)],
    )


@functools.partial(
    pl.kernel,
    out_type=jax.ShapeDtypeStruct((NC, NP), _f32),
    mesh=_mesh,
    scratch_types=[
        pltpu.VMEM((2, G, 128), jnp.int32),
        pltpu.VMEM((1, 128), _f32),
        pltpu.VMEM((STRIPE,), _f32),
        pltpu.VMEM_SHARED((NP,), _f32),
        pltpu.SemaphoreType.DMA,
        pltpu.SemaphoreType.DMA,
    ],
    compiler_params=_sc_params,
)
def _deg_kernel(ei_hbm, out_hbm, dst_v, ones_v, zbuf, acc, ssem, sem):
    cid = lax.axis_index("c")
    sid = lax.axis_index("s")
    u0, nunits = _worker_units(cid, sid)

    def fire_stage(u, p):
        pltpu.async_copy(ei_hbm.at[1, pl.ds((u0 + u) * G, G)], dst_v.at[p],
                         ssem)

    def wait_stage(p):
        pltpu.make_async_copy(ei_hbm.at[1, pl.ds(0, G)], dst_v.at[p],
                              ssem).wait()

    fire_stage(0, 0)

    ones = jnp.ones((L,), _f32)
    for i in range(128 // L):
        ones_v[0, pl.ds(i * L, L)] = ones

    _zero_acc_stripe(zbuf, acc, sid)
    plsc.subcore_barrier()

    def drain(p):
        for j in range(G):
            pltpu.make_async_copy(ones_v.at[0], acc.at[dst_v.at[p, j]],
                                  sem).wait()

    @pl.loop(0, nunits)
    def _(blk):
        p = blk % 2
        wait_stage(p)

        @pl.when(blk > 0)
        def _():
            drain(1 - p)

        @pl.when(blk + 1 < nunits)
        def _():
            fire_stage(blk + 1, 1 - p)

        for j in range(G):
            pltpu.async_copy(ones_v.at[0], acc.at[dst_v.at[p, j]], sem,
                             add=True)

    drain((nunits - 1) % 2)
    plsc.subcore_barrier()
    _copy_out_stripe(acc, out_hbm, cid, sid)


@functools.partial(
    pl.kernel,
    out_type=jax.ShapeDtypeStruct((NC, NP), _f32),
    mesh=_mesh,
    scratch_types=[
        pltpu.VMEM((2, G, 128), jnp.int32),
        pltpu.VMEM((2, G, 128), jnp.int32),
        pltpu.VMEM((2, G, 128), _f32),
        pltpu.VMEM((STRIPE,), _f32),
        pltpu.VMEM((NP,), _f32),
        pltpu.VMEM_SHARED((NP,), _f32),
        pltpu.SemaphoreType.DMA,
        pltpu.SemaphoreType.DMA,
        pltpu.SemaphoreType.DMA,
    ],
    compiler_params=_sc_params,
)
def _segsum_kernel(ei_hbm, val_hbm, out_hbm,
                   src_v, dst_v, msg_v, zbuf, val_v, acc, lsem, ssem, sem):
    cid = lax.axis_index("c")
    sid = lax.axis_index("s")
    u0, nunits = _worker_units(cid, sid)

    def fire_stage(u, p):
        r = (u0 + u) * G
        pltpu.async_copy(ei_hbm.at[0, pl.ds(r, G)], src_v.at[p], ssem)
        pltpu.async_copy(ei_hbm.at[1, pl.ds(r, G)], dst_v.at[p], ssem)

    def wait_stage(p):
        pltpu.make_async_copy(ei_hbm.at[0, pl.ds(0, G)], src_v.at[p],
                              ssem).wait()
        pltpu.make_async_copy(ei_hbm.at[1, pl.ds(0, G)], dst_v.at[p],
                              ssem).wait()

    fire_stage(0, 0)
    load = pltpu.async_copy(val_hbm, val_v, lsem)
    _zero_acc_stripe(zbuf, acc, sid)
    load.wait()
    plsc.subcore_barrier()

    def drain(p):
        for j in range(G):
            pltpu.make_async_copy(msg_v.at[p, j], acc.at[dst_v.at[p, j]],
                                  sem).wait()

    @pl.loop(0, nunits)
    def _(blk):
        p = blk % 2
        wait_stage(p)

        @pl.when(blk > 0)
        def _():
            drain(1 - p)

        @pl.when(blk + 1 < nunits)
        def _():
            fire_stage(blk + 1, 1 - p)

        for j in range(G):
            for i in range(128 // L):
                idx = src_v[p, j, pl.ds(i * L, L)]
                msg_v[p, j, pl.ds(i * L, L)] = plsc.load_gather(val_v, [idx])
            pltpu.async_copy(msg_v.at[p, j], acc.at[dst_v.at[p, j]], sem,
                             add=True)

    drain((nunits - 1) % 2)
    plsc.subcore_barrier()
    _copy_out_stripe(acc, out_hbm, cid, sid)


_R = NP // 128
_BR = 112
_GRID = _R // _BR

_vspec = pl.BlockSpec((_BR, 128), lambda i: (i, 0))


def _sspec():
    return pl.BlockSpec(memory_space=pltpu.SMEM)


def _node1_body(d0, d1, x, dinv, y):
    deg = d0[...] + d1[...] + 1.0
    inv = lax.rsqrt(jnp.maximum(deg, 1.0))
    dinv[...] = inv
    y[...] = inv * x[...]


_node1 = pl.pallas_call(
    _node1_body,
    grid=(_GRID,),
    in_specs=[_vspec, _vspec, _vspec],
    out_specs=[_vspec, _vspec],
    out_shape=[jax.ShapeDtypeStruct((_R, 128), _f32)] * 2,
)


def _node2_body(t0, t1, y, dinv, w1, b1, v, wout):
    s1 = dinv[...] * (t0[...] + t1[...] + y[...])
    acc = jnp.zeros_like(s1)
    for k in range(16):
        acc = acc + jnp.maximum(s1 * w1[0, k] + b1[0, k], 0.0) * v[0, k]
    wout[...] = dinv[...] * acc


_node2 = pl.pallas_call(
    _node2_body,
    grid=(_GRID,),
    in_specs=[_vspec, _vspec, _vspec, _vspec, _sspec(), _sspec(), _sspec()],
    out_specs=pl.BlockSpec((_BR, 128), lambda i: (i, 0)),
    out_shape=jax.ShapeDtypeStruct((_R, 128), _f32),
)


def _node3_body(t0, t1, w, dinv, c0, out):
    out[...] = dinv[...] * (t0[...] + t1[...] + w[...]) + c0[0]


_node3 = pl.pallas_call(
    _node3_body,
    grid=(_GRID,),
    in_specs=[_vspec, _vspec, _vspec, _vspec, _sspec()],
    out_specs=pl.BlockSpec((_BR, 128), lambda i: (i, 0)),
    out_shape=jax.ShapeDtypeStruct((_R, 128), _f32),
)


def kernel(x, edge_index, W1, b1, W2, b2, Wl, bl):
    assert edge_index.shape == (2, E)
    ei3 = edge_index.reshape(2, ROWS, 128)
    xp = jnp.pad(x[:, 0], (0, NP - N))

    degp = _deg_kernel(ei3)
    dinv2d, y2d = _node1(degp[0].reshape(_R, 128), degp[1].reshape(_R, 128),
                         xp.reshape(_R, 128))

    t1p = _segsum_kernel(ei3, y2d.reshape(NP))
    v = (W2 @ Wl).reshape(1, 16)
    w2d = _node2(t1p[0].reshape(_R, 128), t1p[1].reshape(_R, 128),
                 y2d, dinv2d, W1.reshape(1, 16), b1.reshape(1, 16), v)

    t2p = _segsum_kernel(ei3, w2d.reshape(NP))
    c0 = (b2 @ Wl + bl).reshape(1)
    out2d = _node3(t2p[0].reshape(_R, 128), t2p[1].reshape(_R, 128),
                   w2d, dinv2d, c0)

    return out2d.reshape(NP)[:N].reshape(N, 1)
